# Initial kernel scaffold; baseline (speedup 1.0000x reference)
import functools

import jax
import jax.numpy as jnp
from jax import lax
from jax.experimental import pallas as pl
from jax.experimental.pallas import tpu as pltpu
from jax.experimental.pallas import tpu_sc as plsc

N = 10000
D = 128
E = 320000
NC = 2
NS = 16
NW = NC * NS
L = 16
CHUNK = 128
EPC = 79
EPT = EPC * CHUNK
EPAD = NW * EPT
NPAD = 10016
RPT = NPAD // NS
HALF = RPT // 2


def _sc_mesh():
    return plsc.VectorSubcoreMesh(core_axis_name="c", subcore_axis_name="s")


def _deg_body(dst_hbm, out_hbm, idx_v, hist_v, ones_v, sem):
    c = lax.axis_index("c")
    s = lax.axis_index("s")
    wid = c * NS + s
    pltpu.async_copy(dst_hbm.at[wid], idx_v, sem).wait()

    def _zero(i, carry):
        hist_v[pl.ds(i * L, L)] = jnp.zeros((L,), jnp.float32)
        return carry

    lax.fori_loop(0, NPAD // L, _zero, 0, unroll=8)
    ones_v[...] = jnp.ones((L,), jnp.float32)

    def _hist(i, carry):
        j = i // (CHUNK // L)
        k = i % (CHUNK // L)
        idx = idx_v[j, pl.ds(k * L, L)]
        plsc.addupdate_scatter(hist_v, [idx], ones_v[...])
        return carry

    lax.fori_loop(0, EPC * (CHUNK // L), _hist, 0, unroll=4)
    pltpu.async_copy(hist_v, out_hbm.at[wid], sem).wait()


def _deg_kernel(dst_r):
    return pl.kernel(
        _deg_body,
        out_type=jax.ShapeDtypeStruct((NW, NPAD), jnp.float32),
        mesh=_sc_mesh(),
        scratch_types=[
            pltpu.VMEM((EPC, CHUNK), jnp.int32),
            pltpu.VMEM((NPAD,), jnp.float32),
            pltpu.VMEM((L,), jnp.float32),
            pltpu.SemaphoreType.DMA,
        ],
    )(dst_r)


def _agg_body(z_hbm, src_hbm, dst_hbm, out_hbm, si_v, di_v, rows_v, stg_v, acc_sh, sem):
    c = lax.axis_index("c")
    s = lax.axis_index("s")
    wid = c * NS + s
    pltpu.async_copy(src_hbm.at[wid], si_v, sem).wait()
    pltpu.async_copy(dst_hbm.at[wid], di_v, sem).wait()

    def _zero(i, carry):
        stg_v[i // (D // L), pl.ds((i % (D // L)) * L, L)] = jnp.zeros((L,), jnp.float32)
        return carry

    lax.fori_loop(0, HALF * (D // L), _zero, 0, unroll=8)
    pltpu.sync_copy(stg_v, acc_sh.at[pl.ds(s * RPT, HALF)])
    pltpu.sync_copy(stg_v, acc_sh.at[pl.ds(s * RPT + HALF, HALF)])
    plsc.subcore_barrier()

    def _edge(j, carry):
        pltpu.async_copy(z_hbm.at[si_v.at[j]], rows_v, sem).wait()
        pltpu.sync_copy(rows_v, acc_sh.at[di_v.at[j]], add=True)
        return carry

    lax.fori_loop(0, EPC, _edge, 0)
    plsc.subcore_barrier()

    pltpu.sync_copy(acc_sh.at[pl.ds(s * RPT, HALF)], stg_v)
    pltpu.async_copy(stg_v, out_hbm.at[c, pl.ds(s * RPT, HALF)], sem).wait()
    pltpu.sync_copy(acc_sh.at[pl.ds(s * RPT + HALF, HALF)], stg_v)
    pltpu.async_copy(stg_v, out_hbm.at[c, pl.ds(s * RPT + HALF, HALF)], sem).wait()


def _agg_kernel(z_p, src_r, dst_r):
    return pl.kernel(
        _agg_body,
        out_type=jax.ShapeDtypeStruct((NC, NPAD, D), jnp.float32),
        mesh=_sc_mesh(),
        scratch_types=[
            pltpu.VMEM((EPC, CHUNK), jnp.int32),
            pltpu.VMEM((EPC, CHUNK), jnp.int32),
            pltpu.VMEM((CHUNK, D), jnp.float32),
            pltpu.VMEM((HALF, D), jnp.float32),
            pltpu.VMEM_SHARED((NPAD, D), jnp.float32),
            pltpu.SemaphoreType.DMA,
        ],
    )(z_p, src_r, dst_r)


def _prep_body(hist_ref, x0_ref, dis_ref, z0_ref):
    cnt = jnp.sum(hist_ref[...], axis=1, keepdims=True)
    row = lax.broadcasted_iota(jnp.int32, (NPAD, 1), 0)
    dis = jnp.where(row < N, lax.rsqrt(cnt + 1.0), 0.0)
    dis_ref[...] = dis
    z0_ref[...] = dis * x0_ref[...]


def _prep_kernel(hist_t, x0_p):
    return pl.pallas_call(
        _prep_body,
        out_shape=(
            jax.ShapeDtypeStruct((NPAD, 1), jnp.float32),
            jax.ShapeDtypeStruct((NPAD, D), jnp.float32),
        ),
    )(hist_t, x0_p)


def _layer_body(u_ref, z_ref, dis_ref, w_ref, b_ref, x_ref, zn_ref):
    dis = dis_ref[...]
    y = dis * (u_ref[0] + u_ref[1] + z_ref[...])
    x = jnp.dot(y, w_ref[...], preferred_element_type=jnp.float32) + b_ref[...][None, :]
    x_ref[...] = x
    zn_ref[...] = dis * x


def _layer_kernel(u, z_p, dis, w, b):
    return pl.pallas_call(
        _layer_body,
        out_shape=(
            jax.ShapeDtypeStruct((NPAD, D), jnp.float32),
            jax.ShapeDtypeStruct((NPAD, D), jnp.float32),
        ),
    )(u, z_p, dis, w, b)


def _final_body(u_ref, z_ref, dis_ref, w_ref, b_ref, x0_ref, x1_ref, x2_ref, sum_ref):
    y = dis_ref[...] * (u_ref[0] + u_ref[1] + z_ref[...])
    x2 = jnp.dot(y, w_ref[...], preferred_element_type=jnp.float32) + b_ref[...][None, :]
    x2_ref[...] = x2
    sum_ref[...] = x0_ref[...] + x1_ref[...] + x2


def _final_kernel(u, z_p, dis, w, b, x0_p, x1_p):
    return pl.pallas_call(
        _final_body,
        out_shape=(
            jax.ShapeDtypeStruct((NPAD, D), jnp.float32),
            jax.ShapeDtypeStruct((NPAD, D), jnp.float32),
        ),
    )(u, z_p, dis, w, b, x0_p, x1_p)


def kernel(item_emb, W1, b1, W2, b2, edge_index):
    ei = edge_index.astype(jnp.int32)
    pad = jnp.full((EPAD - E,), N, jnp.int32)
    src_r = jnp.concatenate([ei[0], pad]).reshape(NW, EPC, CHUNK)
    dst_r = jnp.concatenate([ei[1], pad]).reshape(NW, EPC, CHUNK)
    x0_p = jnp.pad(item_emb, ((0, NPAD - N), (0, 0)))

    hist = _deg_kernel(dst_r)
    dis, z0 = _prep_kernel(hist.T, x0_p)
    u1 = _agg_kernel(z0, src_r, dst_r)
    x1_p, z1 = _layer_kernel(u1, z0, dis, W1, b1)
    u2 = _agg_kernel(z1, src_r, dst_r)
    x2_p, summed_p = _final_kernel(u2, z1, dis, W2, b2, x0_p, x1_p)

    return (summed_p[:N], item_emb, x1_p[:N], x2_p[:N])


# speedup vs baseline: 9.7566x; 9.7566x over previous
import functools

import jax
import jax.numpy as jnp
from jax import lax
from jax.experimental import pallas as pl
from jax.experimental.pallas import tpu as pltpu
from jax.experimental.pallas import tpu_sc as plsc

N = 10000
D = 128
E = 320000
NC = 2
NS = 16
NW = NC * NS
L = 16
CHUNK = 128
EPC = 79
EPT = EPC * CHUNK
EPAD = NW * EPT
NPAD = 10112
RPT = NPAD // NS


def _sc_mesh():
    return plsc.VectorSubcoreMesh(core_axis_name="c", subcore_axis_name="s")


def _deg_body(dst_hbm, out_hbm, di_v, ones_v, stg_v, deg_sh, sem):
    c = lax.axis_index("c")
    s = lax.axis_index("s")
    wid = c * NS + s

    e0 = jnp.where(lax.iota(jnp.int32, L) == 0, 1.0, 0.0).astype(jnp.float32)

    def _fill(i, carry):
        ones_v[i, :] = e0
        return carry

    lax.fori_loop(0, CHUNK, _fill, 0, unroll=8)

    def _zero(i, carry):
        stg_v[i, :] = jnp.zeros((L,), jnp.float32)
        return carry

    lax.fori_loop(0, RPT, _zero, 0, unroll=8)
    pltpu.sync_copy(stg_v, deg_sh.at[pl.ds(s * RPT, RPT)])
    plsc.subcore_barrier()

    def _hist(j, carry):
        pltpu.async_copy(dst_hbm.at[wid, j], di_v, sem).wait()
        pltpu.sync_copy(ones_v, deg_sh.at[di_v], add=True)
        return carry

    lax.fori_loop(0, EPC, _hist, 0)
    plsc.subcore_barrier()

    pltpu.sync_copy(deg_sh.at[pl.ds(s * RPT, RPT)], stg_v)
    pltpu.async_copy(stg_v, out_hbm.at[c, pl.ds(s * RPT, RPT)], sem).wait()


def _deg_kernel(dst_r):
    return pl.kernel(
        _deg_body,
        out_type=jax.ShapeDtypeStruct((NC, NPAD, L), jnp.float32),
        mesh=_sc_mesh(),
        compiler_params=pltpu.CompilerParams(use_tc_tiling_on_sc=False),
        scratch_types=[
            pltpu.VMEM((CHUNK,), jnp.int32),
            pltpu.VMEM((CHUNK, L), jnp.float32),
            pltpu.VMEM((RPT, L), jnp.float32),
            pltpu.VMEM_SHARED((NPAD, L), jnp.float32),
            pltpu.SemaphoreType.DMA,
        ],
    )(dst_r)


_PIECES = [(0, CHUNK), (CHUNK, CHUNK), (2 * CHUNK, CHUNK), (3 * CHUNK, CHUNK),
           (4 * CHUNK, RPT - 4 * CHUNK)]


def _agg_body(z_hbm, src_hbm, dst_hbm, out_hbm, si_v, di_v, rows_v, acc_sh, sem):
    c = lax.axis_index("c")
    s = lax.axis_index("s")
    wid = c * NS + s

    def _zero(i, carry):
        rows_v[i // (D // L), pl.ds((i % (D // L)) * L, L)] = jnp.zeros((L,), jnp.float32)
        return carry

    lax.fori_loop(0, CHUNK * (D // L), _zero, 0, unroll=8)
    for off, ln in _PIECES:
        pltpu.sync_copy(rows_v.at[pl.ds(0, ln)], acc_sh.at[pl.ds(s * RPT + off, ln)])
    plsc.subcore_barrier()

    def _edge(j, carry):
        pltpu.async_copy(src_hbm.at[wid, j], si_v, sem).wait()
        pltpu.async_copy(dst_hbm.at[wid, j], di_v, sem).wait()
        pltpu.async_copy(z_hbm.at[si_v], rows_v, sem).wait()
        pltpu.sync_copy(rows_v, acc_sh.at[di_v], add=True)
        return carry

    lax.fori_loop(0, EPC, _edge, 0)
    plsc.subcore_barrier()

    for off, ln in _PIECES:
        pltpu.sync_copy(acc_sh.at[pl.ds(s * RPT + off, ln)], rows_v.at[pl.ds(0, ln)])
        pltpu.async_copy(rows_v.at[pl.ds(0, ln)],
                         out_hbm.at[c, pl.ds(s * RPT + off, ln)], sem).wait()


def _agg_kernel(z_p, src_r, dst_r):
    return pl.kernel(
        _agg_body,
        out_type=jax.ShapeDtypeStruct((NC, NPAD, D), jnp.float32),
        mesh=_sc_mesh(),
        scratch_types=[
            pltpu.VMEM((CHUNK,), jnp.int32),
            pltpu.VMEM((CHUNK,), jnp.int32),
            pltpu.VMEM((CHUNK, D), jnp.float32),
            pltpu.VMEM_SHARED((NPAD, D), jnp.float32),
            pltpu.SemaphoreType.DMA,
        ],
    )(z_p, src_r, dst_r)


def _prep_body(hist_ref, x0_ref, dis_ref, z0_ref):
    cnt = hist_ref[0, :, 0:1] + hist_ref[1, :, 0:1]
    row = lax.broadcasted_iota(jnp.int32, (NPAD, 1), 0)
    dis = jnp.where(row < N, lax.rsqrt(cnt + 1.0), 0.0)
    dis_ref[...] = dis
    z0_ref[...] = dis * x0_ref[...]


def _prep_kernel(hist_t, x0_p):
    return pl.pallas_call(
        _prep_body,
        out_shape=(
            jax.ShapeDtypeStruct((NPAD, 1), jnp.float32),
            jax.ShapeDtypeStruct((NPAD, D), jnp.float32),
        ),
    )(hist_t, x0_p)


def _layer_body(u_ref, z_ref, dis_ref, w_ref, b_ref, x_ref, zn_ref):
    dis = dis_ref[...]
    y = dis * (u_ref[0] + u_ref[1] + z_ref[...])
    x = jnp.dot(y, w_ref[...], preferred_element_type=jnp.float32) + b_ref[...][None, :]
    x_ref[...] = x
    zn_ref[...] = dis * x


def _layer_kernel(u, z_p, dis, w, b):
    return pl.pallas_call(
        _layer_body,
        out_shape=(
            jax.ShapeDtypeStruct((NPAD, D), jnp.float32),
            jax.ShapeDtypeStruct((NPAD, D), jnp.float32),
        ),
    )(u, z_p, dis, w, b)


def _final_body(u_ref, z_ref, dis_ref, w_ref, b_ref, x0_ref, x1_ref, x2_ref, sum_ref):
    y = dis_ref[...] * (u_ref[0] + u_ref[1] + z_ref[...])
    x2 = jnp.dot(y, w_ref[...], preferred_element_type=jnp.float32) + b_ref[...][None, :]
    x2_ref[...] = x2
    sum_ref[...] = x0_ref[...] + x1_ref[...] + x2


def _final_kernel(u, z_p, dis, w, b, x0_p, x1_p):
    return pl.pallas_call(
        _final_body,
        out_shape=(
            jax.ShapeDtypeStruct((NPAD, D), jnp.float32),
            jax.ShapeDtypeStruct((NPAD, D), jnp.float32),
        ),
    )(u, z_p, dis, w, b, x0_p, x1_p)


def kernel(item_emb, W1, b1, W2, b2, edge_index):
    ei = edge_index.astype(jnp.int32)
    pad = jnp.full((EPAD - E,), N, jnp.int32)
    src_r = jnp.concatenate([ei[0], pad]).reshape(NW, EPC, CHUNK)
    dst_r = jnp.concatenate([ei[1], pad]).reshape(NW, EPC, CHUNK)
    x0_p = jnp.pad(item_emb, ((0, NPAD - N), (0, 0)))

    hist = _deg_kernel(dst_r)
    dis, z0 = _prep_kernel(hist, x0_p)
    u1 = _agg_kernel(z0, src_r, dst_r)
    x1_p, z1 = _layer_kernel(u1, z0, dis, W1, b1)
    u2 = _agg_kernel(z1, src_r, dst_r)
    x2_p, summed_p = _final_kernel(u2, z1, dis, W2, b2, x0_p, x1_p)

    return (summed_p[:N], item_emb, x1_p[:N], x2_p[:N])
